# baseline (speedup 1.0000x reference)
import functools

import jax
import jax.numpy as jnp
import numpy as np
from jax import lax
from jax.experimental import pallas as pl
from jax.experimental.pallas import tpu as pltpu
from jax.experimental.pallas import tpu_sc as plsc

VOCAB = 100000
MIN_N = 2
MAX_N = 3
NUM_HEADS = 4
TBL = 100000
EDIM = 64
HID = 2048
KSIZE = 3
ORDERS = MAX_N - MIN_N + 1
MEMD = ORDERS * NUM_HEADS * EDIM
BATCH = 2
SEQ = 4096
BS = BATCH * SEQ

_rng = np.random.RandomState(1234)
_HASH_MULT = ((_rng.randint(1, 2**31 - 1, size=(NUM_HEADS, MAX_N)) * 2 + 1)
              % (2**32)).astype(np.uint32)

NW = 32
CHUNK = BS // NW
GSUB = 128
NSUB = CHUNK // GSUB


_RB = 512


def _tp_body(tin_ref, out_ref):
    t = tin_ref[0]
    tt = jnp.transpose(t)
    out_ref[0] = jnp.concatenate(
        [tt, jnp.zeros((_RB, EDIM), jnp.float32)], axis=1)


def _tc_transpose_pad(tables_t):
    return pl.pallas_call(
        _tp_body,
        grid=(ORDERS * NUM_HEADS, pl.cdiv(TBL, _RB)),
        in_specs=[pl.BlockSpec((1, EDIM, _RB), lambda k, j: (k, 0, j))],
        out_specs=pl.BlockSpec((1, _RB, 2 * EDIM), lambda k, j: (k, j, 0)),
        out_shape=jax.ShapeDtypeStruct((ORDERS * NUM_HEADS, TBL, 2 * EDIM),
                                       jnp.float32),
    )(tables_t)


def _sc_gather_mem(tables_flat, idt, idm1, idm2):
    mesh = plsc.VectorSubcoreMesh(core_axis_name="c", subcore_axis_name="s")

    @functools.partial(
        pl.kernel,
        mesh=mesh,
        out_type=jax.ShapeDtypeStruct((ORDERS * NUM_HEADS, BS, 2 * EDIM),
                                      jnp.float32),
        scratch_types=[
            pltpu.VMEM((CHUNK,), jnp.int32),
            pltpu.VMEM((CHUNK,), jnp.int32),
            pltpu.VMEM((CHUNK,), jnp.int32),
            pltpu.VMEM((NSUB, GSUB), jnp.int32),
            pltpu.VMEM((CHUNK, 2 * EDIM), jnp.float32),
            pltpu.SemaphoreType.DMA,
        ],
    )
    def gk(tbl_hbm, idt_hbm, idm1_hbm, idm2_hbm, out_hbm,
           idt_v, idm1_v, idm2_v, idx_v, rows_v, sem):
        wid = lax.axis_index("s") * 2 + lax.axis_index("c")
        base = wid * CHUNK
        pltpu.sync_copy(idt_hbm.at[pl.ds(base, CHUNK)], idt_v)
        pltpu.sync_copy(idm1_hbm.at[pl.ds(base, CHUNK)], idm1_v)
        pltpu.sync_copy(idm2_hbm.at[pl.ds(base, CHUNK)], idm2_v)

        for o, n in enumerate(range(MIN_N, MAX_N + 1)):
            for h in range(NUM_HEADS):
                kk = o * NUM_HEADS + h
                m = _HASH_MULT[h]
                for v in range(CHUNK // 16):
                    sl = pl.ds(v * 16, 16)
                    t0 = plsc.bitcast(idt_v[sl], jnp.uint32)
                    t1 = plsc.bitcast(idm1_v[sl], jnp.uint32)
                    if n == 2:
                        acc = t1 * jnp.uint32(m[0]) + t0 * jnp.uint32(m[1])
                    else:
                        t2 = plsc.bitcast(idm2_v[sl], jnp.uint32)
                        acc = (t2 * jnp.uint32(m[0]) + t1 * jnp.uint32(m[1])
                               + t0 * jnp.uint32(m[2]))
                    acc = acc ^ (acc >> jnp.uint32(16))
                    acc = acc % jnp.uint32(TBL)
                    row = plsc.bitcast(acc, jnp.int32)
                    idx_v[v // (GSUB // 16), pl.ds((v % (GSUB // 16)) * 16, 16)] = row
                cps = [
                    pltpu.async_copy(
                        tbl_hbm.at[kk].at[idx_v.at[c]],
                        rows_v.at[pl.ds(c * GSUB, GSUB)],
                        sem,
                    )
                    for c in range(NSUB)
                ]
                for cp in cps:
                    cp.wait()
                pltpu.sync_copy(rows_v, out_hbm.at[kk, pl.ds(base, CHUNK)])

    return gk(tables_flat, idt, idm1, idm2)


_TBLK = 512


def _tc_body(hs_ref, mem_ref, wg_ref, wv_ref, bg_ref, bv_ref, cw_ref, cb_ref,
             out_ref, carry_ref):
    j = pl.program_id(1)
    hs = hs_ref[0]
    mem = jnp.concatenate(
        [mem_ref[kk][:, :EDIM] for kk in range(ORDERS * NUM_HEADS)], axis=1
    )
    memh = mem.astype(jnp.bfloat16)
    keyv = jnp.dot(hs.astype(jnp.bfloat16), wg_ref[...],
                   preferred_element_type=jnp.float32) + bg_ref[...]
    dot = jnp.sum(keyv * mem, axis=1, keepdims=True) * (1.0 / np.sqrt(MEMD))
    alpha = 1.0 / (1.0 + jnp.exp(-dot))
    value = jnp.dot(memh, wv_ref[...],
                    preferred_element_type=jnp.float32) + bv_ref[...]
    gated = alpha * value
    prev = jnp.where(j == 0, 0.0, carry_ref[0:2])
    g_m1 = jnp.concatenate([prev[1:2], gated[:-1]], axis=0)
    g_m2 = jnp.concatenate([prev[0:2], gated[:-2]], axis=0)
    fused = (g_m2 * cw_ref[0:1] + g_m1 * cw_ref[1:2] + gated * cw_ref[2:3]
             + cb_ref[...])
    out_ref[0] = hs + fused
    carry_ref[0:2] = gated[_TBLK - 2:]


def _tc_dense(hs, mem3, Wg, bg, Wv, bv, conv_w, conv_b):
    grid = (BATCH, SEQ // _TBLK)
    return pl.pallas_call(
        _tc_body,
        grid=grid,
        in_specs=[
            pl.BlockSpec((1, _TBLK, HID), lambda b, j: (b, j, 0)),
            pl.BlockSpec((ORDERS * NUM_HEADS, _TBLK, 2 * EDIM),
                         lambda b, j: (0, b * (SEQ // _TBLK) + j, 0)),
            pl.BlockSpec((HID, MEMD), lambda b, j: (0, 0)),
            pl.BlockSpec((MEMD, HID), lambda b, j: (0, 0)),
            pl.BlockSpec((1, MEMD), lambda b, j: (0, 0)),
            pl.BlockSpec((1, HID), lambda b, j: (0, 0)),
            pl.BlockSpec((KSIZE, HID), lambda b, j: (0, 0)),
            pl.BlockSpec((1, HID), lambda b, j: (0, 0)),
        ],
        out_specs=pl.BlockSpec((1, _TBLK, HID), lambda b, j: (b, j, 0)),
        out_shape=jax.ShapeDtypeStruct((BATCH, SEQ, HID), jnp.float32),
        scratch_shapes=[pltpu.VMEM((8, HID), jnp.float32)],
        compiler_params=pltpu.CompilerParams(
            dimension_semantics=("arbitrary", "arbitrary"),
        ),
    )(hs, mem3, Wg.astype(jnp.bfloat16), Wv.astype(jnp.bfloat16),
      bg.reshape(1, MEMD), bv.reshape(1, HID),
      conv_w.T, conv_b.reshape(1, HID))


def kernel(hidden_states, input_ids, tables, Wg, bg, Wv, bv, conv_w, conv_b):
    ids = input_ids.astype(jnp.int32)
    idm1 = jnp.pad(ids, ((0, 0), (1, 0)))[:, :SEQ]
    idm2 = jnp.pad(ids, ((0, 0), (2, 0)))[:, :SEQ]
    tables_t = jnp.transpose(tables, (0, 2, 1))
    tables_pad = _tc_transpose_pad(tables_t)
    mem = _sc_gather_mem(tables_pad, ids.reshape(BS), idm1.reshape(BS),
                         idm2.reshape(BS))
    return _tc_dense(hidden_states, mem, Wg, bg, Wv, bv, conv_w, conv_b)


# speedup vs baseline: 2.1155x; 2.1155x over previous
import functools

import jax
import jax.numpy as jnp
import numpy as np
from jax import lax
from jax.experimental import pallas as pl
from jax.experimental.pallas import tpu as pltpu
from jax.experimental.pallas import tpu_sc as plsc

VOCAB = 100000
MIN_N = 2
MAX_N = 3
NUM_HEADS = 4
TBL = 100000
EDIM = 64
HID = 2048
KSIZE = 3
ORDERS = MAX_N - MIN_N + 1
MEMD = ORDERS * NUM_HEADS * EDIM
BATCH = 2
SEQ = 4096
BS = BATCH * SEQ

_rng = np.random.RandomState(1234)
_HASH_MULT = ((_rng.randint(1, 2**31 - 1, size=(NUM_HEADS, MAX_N)) * 2 + 1)
              % (2**32)).astype(np.uint32)

NW = 32
CHUNK = BS // NW
GSUB = 128
NSUB = CHUNK // GSUB


_RB = 2048


def _tp_body(tin_ref, out_ref):
    t = tin_ref[0]
    eye = jnp.eye(EDIM, dtype=jnp.float32)
    tt = lax.dot_general(t, eye, (((0,), (0,)), ((), ())),
                         preferred_element_type=jnp.float32)
    out_ref[0] = jnp.concatenate(
        [tt, jnp.zeros((_RB, EDIM), jnp.float32)], axis=1)


def _tc_transpose_pad(tables_t):
    return pl.pallas_call(
        _tp_body,
        grid=(ORDERS * NUM_HEADS, pl.cdiv(TBL, _RB)),
        in_specs=[pl.BlockSpec((1, EDIM, _RB), lambda k, j: (k, 0, j))],
        out_specs=pl.BlockSpec((1, _RB, 2 * EDIM), lambda k, j: (k, j, 0)),
        out_shape=jax.ShapeDtypeStruct((ORDERS * NUM_HEADS, TBL, 2 * EDIM),
                                       jnp.float32),
    )(tables_t)


def _sc_gather_mem(tables_flat, idt, idm1, idm2):
    mesh = plsc.VectorSubcoreMesh(core_axis_name="c", subcore_axis_name="s")

    @functools.partial(
        pl.kernel,
        mesh=mesh,
        out_type=jax.ShapeDtypeStruct((ORDERS * NUM_HEADS, BS, 2 * EDIM),
                                      jnp.float32),
        scratch_types=[
            pltpu.VMEM((CHUNK,), jnp.int32),
            pltpu.VMEM((CHUNK,), jnp.int32),
            pltpu.VMEM((CHUNK,), jnp.int32),
            pltpu.VMEM((NSUB, GSUB), jnp.int32),
            pltpu.VMEM((CHUNK, 2 * EDIM), jnp.float32),
            pltpu.SemaphoreType.DMA,
        ],
    )
    def gk(tbl_hbm, idt_hbm, idm1_hbm, idm2_hbm, out_hbm,
           idt_v, idm1_v, idm2_v, idx_v, rows_v, sem):
        wid = lax.axis_index("s") * 2 + lax.axis_index("c")
        base = wid * CHUNK
        pltpu.sync_copy(idt_hbm.at[pl.ds(base, CHUNK)], idt_v)
        pltpu.sync_copy(idm1_hbm.at[pl.ds(base, CHUNK)], idm1_v)
        pltpu.sync_copy(idm2_hbm.at[pl.ds(base, CHUNK)], idm2_v)

        for o, n in enumerate(range(MIN_N, MAX_N + 1)):
            for h in range(NUM_HEADS):
                kk = o * NUM_HEADS + h
                m = _HASH_MULT[h]
                for v in range(CHUNK // 16):
                    sl = pl.ds(v * 16, 16)
                    t0 = plsc.bitcast(idt_v[sl], jnp.uint32)
                    t1 = plsc.bitcast(idm1_v[sl], jnp.uint32)
                    if n == 2:
                        acc = t1 * jnp.uint32(m[0]) + t0 * jnp.uint32(m[1])
                    else:
                        t2 = plsc.bitcast(idm2_v[sl], jnp.uint32)
                        acc = (t2 * jnp.uint32(m[0]) + t1 * jnp.uint32(m[1])
                               + t0 * jnp.uint32(m[2]))
                    acc = acc ^ (acc >> jnp.uint32(16))
                    acc = acc % jnp.uint32(TBL)
                    row = plsc.bitcast(acc, jnp.int32)
                    idx_v[v // (GSUB // 16), pl.ds((v % (GSUB // 16)) * 16, 16)] = row
                cps = [
                    pltpu.async_copy(
                        tbl_hbm.at[kk].at[idx_v.at[c]],
                        rows_v.at[pl.ds(c * GSUB, GSUB)],
                        sem,
                    )
                    for c in range(NSUB)
                ]
                for cp in cps:
                    cp.wait()
                pltpu.sync_copy(rows_v, out_hbm.at[kk, pl.ds(base, CHUNK)])

    return gk(tables_flat, idt, idm1, idm2)


_TBLK = 512


def _tc_body(hs_ref, mem_ref, wg_ref, wv_ref, bg_ref, bv_ref, cw_ref, cb_ref,
             out_ref, carry_ref):
    j = pl.program_id(1)
    hs = hs_ref[0]
    mem = jnp.concatenate(
        [mem_ref[kk][:, :EDIM] for kk in range(ORDERS * NUM_HEADS)], axis=1
    )
    memh = mem.astype(jnp.bfloat16)
    keyv = jnp.dot(hs.astype(jnp.bfloat16), wg_ref[...],
                   preferred_element_type=jnp.float32) + bg_ref[...]
    dot = jnp.sum(keyv * mem, axis=1, keepdims=True) * (1.0 / np.sqrt(MEMD))
    alpha = 1.0 / (1.0 + jnp.exp(-dot))
    value = jnp.dot(memh, wv_ref[...],
                    preferred_element_type=jnp.float32) + bv_ref[...]
    gated = alpha * value
    prev = jnp.where(j == 0, 0.0, carry_ref[0:2])
    g_m1 = jnp.concatenate([prev[1:2], gated[:-1]], axis=0)
    g_m2 = jnp.concatenate([prev[0:2], gated[:-2]], axis=0)
    fused = (g_m2 * cw_ref[0:1] + g_m1 * cw_ref[1:2] + gated * cw_ref[2:3]
             + cb_ref[...])
    out_ref[0] = hs + fused
    carry_ref[0:2] = gated[_TBLK - 2:]


def _tc_dense(hs, mem3, Wg, bg, Wv, bv, conv_w, conv_b):
    grid = (BATCH, SEQ // _TBLK)
    return pl.pallas_call(
        _tc_body,
        grid=grid,
        in_specs=[
            pl.BlockSpec((1, _TBLK, HID), lambda b, j: (b, j, 0)),
            pl.BlockSpec((ORDERS * NUM_HEADS, _TBLK, 2 * EDIM),
                         lambda b, j: (0, b * (SEQ // _TBLK) + j, 0)),
            pl.BlockSpec((HID, MEMD), lambda b, j: (0, 0)),
            pl.BlockSpec((MEMD, HID), lambda b, j: (0, 0)),
            pl.BlockSpec((1, MEMD), lambda b, j: (0, 0)),
            pl.BlockSpec((1, HID), lambda b, j: (0, 0)),
            pl.BlockSpec((KSIZE, HID), lambda b, j: (0, 0)),
            pl.BlockSpec((1, HID), lambda b, j: (0, 0)),
        ],
        out_specs=pl.BlockSpec((1, _TBLK, HID), lambda b, j: (b, j, 0)),
        out_shape=jax.ShapeDtypeStruct((BATCH, SEQ, HID), jnp.float32),
        scratch_shapes=[pltpu.VMEM((8, HID), jnp.float32)],
        compiler_params=pltpu.CompilerParams(
            dimension_semantics=("arbitrary", "arbitrary"),
        ),
    )(hs, mem3, Wg.astype(jnp.bfloat16), Wv.astype(jnp.bfloat16),
      bg.reshape(1, MEMD), bv.reshape(1, HID),
      conv_w.T, conv_b.reshape(1, HID))


def kernel(hidden_states, input_ids, tables, Wg, bg, Wv, bv, conv_w, conv_b):
    ids = input_ids.astype(jnp.int32)
    idm1 = jnp.pad(ids, ((0, 0), (1, 0)))[:, :SEQ]
    idm2 = jnp.pad(ids, ((0, 0), (2, 0)))[:, :SEQ]
    tables_t = jnp.transpose(tables, (0, 2, 1))
    tables_pad = _tc_transpose_pad(tables_t)
    mem = _sc_gather_mem(tables_pad, ids.reshape(BS), idm1.reshape(BS),
                         idm2.reshape(BS))
    return _tc_dense(hidden_states, mem, Wg, bg, Wv, bv, conv_w, conv_b)


# speedup vs baseline: 2.5369x; 1.1992x over previous
import functools

import jax
import jax.numpy as jnp
import numpy as np
from jax import lax
from jax.experimental import pallas as pl
from jax.experimental.pallas import tpu as pltpu
from jax.experimental.pallas import tpu_sc as plsc

VOCAB = 100000
MIN_N = 2
MAX_N = 3
NUM_HEADS = 4
TBL = 100000
EDIM = 64
HID = 2048
KSIZE = 3
ORDERS = MAX_N - MIN_N + 1
MEMD = ORDERS * NUM_HEADS * EDIM
BATCH = 2
SEQ = 4096
BS = BATCH * SEQ

_rng = np.random.RandomState(1234)
_HASH_MULT = ((_rng.randint(1, 2**31 - 1, size=(NUM_HEADS, MAX_N)) * 2 + 1)
              % (2**32)).astype(np.uint32)

NW = 32
CHUNK = BS // NW
GSUB = 128
NSUB = CHUNK // GSUB


_RB = 2048


def _tp_body(tin_ref, out_ref):
    eye = jnp.eye(EDIM, dtype=jnp.float32)
    packed = []
    for i in range(4):
        t = tin_ref[i]
        tt = lax.dot_general(t, eye, (((0,), (0,)), ((), ())),
                             preferred_element_type=jnp.float32)
        b = lax.bitcast_convert_type(tt, jnp.uint32)
        w = (b[:, EDIM // 2:] & jnp.uint32(0xFFFF0000)) | (b[:, :EDIM // 2]
                                                          >> jnp.uint32(16))
        packed.append(lax.bitcast_convert_type(w, jnp.int32))
    out_ref[0] = jnp.concatenate(packed, axis=1)


def _tc_transpose_pad(tables_t):
    return pl.pallas_call(
        _tp_body,
        grid=(2, pl.cdiv(TBL, _RB)),
        in_specs=[pl.BlockSpec((4, EDIM, _RB), lambda g, j: (g, 0, j))],
        out_specs=pl.BlockSpec((1, _RB, 2 * EDIM), lambda g, j: (g, j, 0)),
        out_shape=jax.ShapeDtypeStruct((2, TBL, 2 * EDIM), jnp.int32),
    )(tables_t)


def _sc_gather_mem(tables_flat, idt, idm1, idm2):
    mesh = plsc.VectorSubcoreMesh(core_axis_name="c", subcore_axis_name="s")

    @functools.partial(
        pl.kernel,
        mesh=mesh,
        out_type=jax.ShapeDtypeStruct((ORDERS * NUM_HEADS, BS, 2 * EDIM),
                                      jnp.int32),
        scratch_types=[
            pltpu.VMEM((CHUNK,), jnp.int32),
            pltpu.VMEM((CHUNK,), jnp.int32),
            pltpu.VMEM((CHUNK,), jnp.int32),
            pltpu.VMEM((NSUB, GSUB), jnp.int32),
            pltpu.VMEM((CHUNK, 2 * EDIM), jnp.int32),
            pltpu.SemaphoreType.DMA,
        ],
    )
    def gk(tbl_hbm, idt_hbm, idm1_hbm, idm2_hbm, out_hbm,
           idt_v, idm1_v, idm2_v, idx_v, rows_v, sem):
        wid = lax.axis_index("s") * 2 + lax.axis_index("c")
        base = wid * CHUNK
        pltpu.sync_copy(idt_hbm.at[pl.ds(base, CHUNK)], idt_v)
        pltpu.sync_copy(idm1_hbm.at[pl.ds(base, CHUNK)], idm1_v)
        pltpu.sync_copy(idm2_hbm.at[pl.ds(base, CHUNK)], idm2_v)

        for o, n in enumerate(range(MIN_N, MAX_N + 1)):
            for h in range(NUM_HEADS):
                kk = o * NUM_HEADS + h
                m = _HASH_MULT[h]
                for v in range(CHUNK // 16):
                    sl = pl.ds(v * 16, 16)
                    t0 = plsc.bitcast(idt_v[sl], jnp.uint32)
                    t1 = plsc.bitcast(idm1_v[sl], jnp.uint32)
                    if n == 2:
                        acc = t1 * jnp.uint32(m[0]) + t0 * jnp.uint32(m[1])
                    else:
                        t2 = plsc.bitcast(idm2_v[sl], jnp.uint32)
                        acc = (t2 * jnp.uint32(m[0]) + t1 * jnp.uint32(m[1])
                               + t0 * jnp.uint32(m[2]))
                    acc = acc ^ (acc >> jnp.uint32(16))
                    acc = acc % jnp.uint32(TBL)
                    row = plsc.bitcast(acc, jnp.int32)
                    idx_v[v // (GSUB // 16), pl.ds((v % (GSUB // 16)) * 16, 16)] = row
                cps = [
                    pltpu.async_copy(
                        tbl_hbm.at[kk // 4].at[idx_v.at[c]],
                        rows_v.at[pl.ds(c * GSUB, GSUB)],
                        sem,
                    )
                    for c in range(NSUB)
                ]
                for cp in cps:
                    cp.wait()
                pltpu.sync_copy(rows_v, out_hbm.at[kk, pl.ds(base, CHUNK)])

    return gk(tables_flat, idt, idm1, idm2)


_TBLK = 512


def _tc_body(hs_ref, mem_ref, wg_ref, wv_ref, bg_ref, bv_ref, cw_ref, cb_ref,
             out_ref, carry_ref):
    j = pl.program_id(1)
    hs = hs_ref[0]
    cols = []
    for kk in range(ORDERS * NUM_HEADS):
        l = kk % 4
        x = mem_ref[kk][:, l * (EDIM // 2):(l + 1) * (EDIM // 2)]
        u = lax.bitcast_convert_type(x, jnp.uint32)
        lo = lax.bitcast_convert_type(u << jnp.uint32(16), jnp.float32)
        hi = lax.bitcast_convert_type(u & jnp.uint32(0xFFFF0000), jnp.float32)
        cols.append(lo)
        cols.append(hi)
    mem = jnp.concatenate(cols, axis=1)
    memh = mem.astype(jnp.bfloat16)
    keyv = jnp.dot(hs.astype(jnp.bfloat16), wg_ref[...],
                   preferred_element_type=jnp.float32) + bg_ref[...]
    dot = jnp.sum(keyv * mem, axis=1, keepdims=True) * (1.0 / np.sqrt(MEMD))
    alpha = 1.0 / (1.0 + jnp.exp(-dot))
    value = jnp.dot(memh, wv_ref[...],
                    preferred_element_type=jnp.float32) + bv_ref[...]
    gated = alpha * value
    prev = jnp.where(j == 0, 0.0, carry_ref[0:2])
    g_m1 = jnp.concatenate([prev[1:2], gated[:-1]], axis=0)
    g_m2 = jnp.concatenate([prev[0:2], gated[:-2]], axis=0)
    fused = (g_m2 * cw_ref[0:1] + g_m1 * cw_ref[1:2] + gated * cw_ref[2:3]
             + cb_ref[...])
    out_ref[0] = hs + fused
    carry_ref[0:2] = gated[_TBLK - 2:]


def _tc_dense(hs, mem3, Wg, bg, Wv, bv, conv_w, conv_b):
    grid = (BATCH, SEQ // _TBLK)
    return pl.pallas_call(
        _tc_body,
        grid=grid,
        in_specs=[
            pl.BlockSpec((1, _TBLK, HID), lambda b, j: (b, j, 0)),
            pl.BlockSpec((ORDERS * NUM_HEADS, _TBLK, 2 * EDIM),
                         lambda b, j: (0, b * (SEQ // _TBLK) + j, 0)),
            pl.BlockSpec((HID, MEMD), lambda b, j: (0, 0)),
            pl.BlockSpec((MEMD, HID), lambda b, j: (0, 0)),
            pl.BlockSpec((1, MEMD), lambda b, j: (0, 0)),
            pl.BlockSpec((1, HID), lambda b, j: (0, 0)),
            pl.BlockSpec((KSIZE, HID), lambda b, j: (0, 0)),
            pl.BlockSpec((1, HID), lambda b, j: (0, 0)),
        ],
        out_specs=pl.BlockSpec((1, _TBLK, HID), lambda b, j: (b, j, 0)),
        out_shape=jax.ShapeDtypeStruct((BATCH, SEQ, HID), jnp.float32),
        scratch_shapes=[pltpu.VMEM((8, HID), jnp.float32)],
        compiler_params=pltpu.CompilerParams(
            dimension_semantics=("arbitrary", "arbitrary"),
        ),
    )(hs, mem3, Wg.astype(jnp.bfloat16), Wv.astype(jnp.bfloat16),
      bg.reshape(1, MEMD), bv.reshape(1, HID),
      conv_w.T, conv_b.reshape(1, HID))


def kernel(hidden_states, input_ids, tables, Wg, bg, Wv, bv, conv_w, conv_b):
    ids = input_ids.astype(jnp.int32)
    idm1 = jnp.pad(ids, ((0, 0), (1, 0)))[:, :SEQ]
    idm2 = jnp.pad(ids, ((0, 0), (2, 0)))[:, :SEQ]
    tables_t = jnp.transpose(tables, (0, 2, 1))
    tables_pad = _tc_transpose_pad(tables_t)
    mem = _sc_gather_mem(tables_pad, ids.reshape(BS), idm1.reshape(BS),
                         idm2.reshape(BS))
    return _tc_dense(hidden_states, mem, Wg, bg, Wv, bv, conv_w, conv_b)


# speedup vs baseline: 2.8856x; 1.1374x over previous
import functools

import jax
import jax.numpy as jnp
import numpy as np
from jax import lax
from jax.experimental import pallas as pl
from jax.experimental.pallas import tpu as pltpu
from jax.experimental.pallas import tpu_sc as plsc

VOCAB = 100000
MIN_N = 2
MAX_N = 3
NUM_HEADS = 4
TBL = 100000
EDIM = 64
HID = 2048
KSIZE = 3
ORDERS = MAX_N - MIN_N + 1
MEMD = ORDERS * NUM_HEADS * EDIM
BATCH = 2
SEQ = 4096
BS = BATCH * SEQ

_rng = np.random.RandomState(1234)
_HASH_MULT = ((_rng.randint(1, 2**31 - 1, size=(NUM_HEADS, MAX_N)) * 2 + 1)
              % (2**32)).astype(np.uint32)

NW = 32
CHUNK = BS // NW
GSUB = 128
NSUB = CHUNK // GSUB


_RB = 4096


def _tp_body(tin_ref, out_ref):
    eye = jnp.eye(EDIM, dtype=jnp.bfloat16)
    packed = []
    for i in range(4):
        t = tin_ref[i].astype(jnp.bfloat16)
        tt = lax.dot_general(t, eye, (((0,), (0,)), ((), ())),
                             preferred_element_type=jnp.float32)
        b = lax.bitcast_convert_type(tt, jnp.uint32)
        w = (b[:, EDIM // 2:] & jnp.uint32(0xFFFF0000)) | (b[:, :EDIM // 2]
                                                          >> jnp.uint32(16))
        packed.append(lax.bitcast_convert_type(w, jnp.int32))
    out_ref[0] = jnp.concatenate(packed, axis=1)


def _tc_transpose_pad(tables_t):
    return pl.pallas_call(
        _tp_body,
        grid=(2, pl.cdiv(TBL, _RB)),
        in_specs=[pl.BlockSpec((4, EDIM, _RB), lambda g, j: (g, 0, j))],
        out_specs=pl.BlockSpec((1, _RB, 2 * EDIM), lambda g, j: (g, j, 0)),
        out_shape=jax.ShapeDtypeStruct((2, TBL, 2 * EDIM), jnp.int32),
    )(tables_t)


def _sc_gather_mem(tables_flat, idt, idm1, idm2):
    mesh = plsc.VectorSubcoreMesh(core_axis_name="c", subcore_axis_name="s")

    @functools.partial(
        pl.kernel,
        mesh=mesh,
        out_type=jax.ShapeDtypeStruct((ORDERS * NUM_HEADS, BS, 2 * EDIM),
                                      jnp.int32),
        scratch_types=[
            pltpu.VMEM((CHUNK,), jnp.int32),
            pltpu.VMEM((CHUNK,), jnp.int32),
            pltpu.VMEM((CHUNK,), jnp.int32),
            pltpu.VMEM((NSUB, GSUB), jnp.int32),
            pltpu.VMEM((CHUNK, 2 * EDIM), jnp.int32),
            pltpu.SemaphoreType.DMA,
        ],
    )
    def gk(tbl_hbm, idt_hbm, idm1_hbm, idm2_hbm, out_hbm,
           idt_v, idm1_v, idm2_v, idx_v, rows_v, sem):
        wid = lax.axis_index("s") * 2 + lax.axis_index("c")
        base = wid * CHUNK
        pltpu.sync_copy(idt_hbm.at[pl.ds(base, CHUNK)], idt_v)
        pltpu.sync_copy(idm1_hbm.at[pl.ds(base, CHUNK)], idm1_v)
        pltpu.sync_copy(idm2_hbm.at[pl.ds(base, CHUNK)], idm2_v)

        for o, n in enumerate(range(MIN_N, MAX_N + 1)):
            for h in range(NUM_HEADS):
                kk = o * NUM_HEADS + h
                m = _HASH_MULT[h]
                for v in range(CHUNK // 16):
                    sl = pl.ds(v * 16, 16)
                    t0 = plsc.bitcast(idt_v[sl], jnp.uint32)
                    t1 = plsc.bitcast(idm1_v[sl], jnp.uint32)
                    if n == 2:
                        acc = t1 * jnp.uint32(m[0]) + t0 * jnp.uint32(m[1])
                    else:
                        t2 = plsc.bitcast(idm2_v[sl], jnp.uint32)
                        acc = (t2 * jnp.uint32(m[0]) + t1 * jnp.uint32(m[1])
                               + t0 * jnp.uint32(m[2]))
                    acc = acc ^ (acc >> jnp.uint32(16))
                    acc = acc % jnp.uint32(TBL)
                    row = plsc.bitcast(acc, jnp.int32)
                    idx_v[v // (GSUB // 16), pl.ds((v % (GSUB // 16)) * 16, 16)] = row
                cps = [
                    pltpu.async_copy(
                        tbl_hbm.at[kk // 4].at[idx_v.at[c]],
                        rows_v.at[pl.ds(c * GSUB, GSUB)],
                        sem,
                    )
                    for c in range(NSUB)
                ]
                for cp in cps:
                    cp.wait()
                pltpu.sync_copy(rows_v, out_hbm.at[kk, pl.ds(base, CHUNK)])

    return gk(tables_flat, idt, idm1, idm2)


_TBLK = 512


def _tc_body(hs_ref, mem_ref, wg_ref, wv_ref, bg_ref, bv_ref, cw_ref, cb_ref,
             out_ref, carry_ref):
    j = pl.program_id(1)
    hs = hs_ref[0]
    cols = []
    for kk in range(ORDERS * NUM_HEADS):
        l = kk % 4
        x = mem_ref[kk][:, l * (EDIM // 2):(l + 1) * (EDIM // 2)]
        u = lax.bitcast_convert_type(x, jnp.uint32)
        lo = lax.bitcast_convert_type(u << jnp.uint32(16), jnp.float32)
        hi = lax.bitcast_convert_type(u & jnp.uint32(0xFFFF0000), jnp.float32)
        cols.append(lo)
        cols.append(hi)
    mem = jnp.concatenate(cols, axis=1)
    memh = mem.astype(jnp.bfloat16)
    keyv = jnp.dot(hs.astype(jnp.bfloat16), wg_ref[...],
                   preferred_element_type=jnp.float32) + bg_ref[...]
    dot = jnp.sum(keyv * mem, axis=1, keepdims=True) * (1.0 / np.sqrt(MEMD))
    alpha = 1.0 / (1.0 + jnp.exp(-dot))
    value = jnp.dot(memh, wv_ref[...],
                    preferred_element_type=jnp.float32) + bv_ref[...]
    gated = alpha * value
    prev = jnp.where(j == 0, 0.0, carry_ref[0:2])
    g_m1 = jnp.concatenate([prev[1:2], gated[:-1]], axis=0)
    g_m2 = jnp.concatenate([prev[0:2], gated[:-2]], axis=0)
    fused = (g_m2 * cw_ref[0:1] + g_m1 * cw_ref[1:2] + gated * cw_ref[2:3]
             + cb_ref[...])
    out_ref[0] = hs + fused
    carry_ref[0:2] = gated[_TBLK - 2:]


def _tc_dense(hs, mem3, Wg, bg, Wv, bv, conv_w, conv_b):
    grid = (BATCH, SEQ // _TBLK)
    return pl.pallas_call(
        _tc_body,
        grid=grid,
        in_specs=[
            pl.BlockSpec((1, _TBLK, HID), lambda b, j: (b, j, 0)),
            pl.BlockSpec((ORDERS * NUM_HEADS, _TBLK, 2 * EDIM),
                         lambda b, j: (0, b * (SEQ // _TBLK) + j, 0)),
            pl.BlockSpec((HID, MEMD), lambda b, j: (0, 0)),
            pl.BlockSpec((MEMD, HID), lambda b, j: (0, 0)),
            pl.BlockSpec((1, MEMD), lambda b, j: (0, 0)),
            pl.BlockSpec((1, HID), lambda b, j: (0, 0)),
            pl.BlockSpec((KSIZE, HID), lambda b, j: (0, 0)),
            pl.BlockSpec((1, HID), lambda b, j: (0, 0)),
        ],
        out_specs=pl.BlockSpec((1, _TBLK, HID), lambda b, j: (b, j, 0)),
        out_shape=jax.ShapeDtypeStruct((BATCH, SEQ, HID), jnp.float32),
        scratch_shapes=[pltpu.VMEM((8, HID), jnp.float32)],
        compiler_params=pltpu.CompilerParams(
            dimension_semantics=("arbitrary", "arbitrary"),
        ),
    )(hs, mem3, Wg.astype(jnp.bfloat16), Wv.astype(jnp.bfloat16),
      bg.reshape(1, MEMD), bv.reshape(1, HID),
      conv_w.T, conv_b.reshape(1, HID))


def kernel(hidden_states, input_ids, tables, Wg, bg, Wv, bv, conv_w, conv_b):
    ids = input_ids.astype(jnp.int32)
    idm1 = jnp.pad(ids, ((0, 0), (1, 0)))[:, :SEQ]
    idm2 = jnp.pad(ids, ((0, 0), (2, 0)))[:, :SEQ]
    tables_t = jnp.transpose(tables, (0, 2, 1))
    tables_pad = _tc_transpose_pad(tables_t)
    mem = _sc_gather_mem(tables_pad, ids.reshape(BS), idm1.reshape(BS),
                         idm2.reshape(BS))
    return _tc_dense(hidden_states, mem, Wg, bg, Wv, bv, conv_w, conv_b)


# speedup vs baseline: 3.5526x; 1.2312x over previous
import functools

import jax
import jax.numpy as jnp
import numpy as np
from jax import lax
from jax.experimental import pallas as pl
from jax.experimental.pallas import tpu as pltpu
from jax.experimental.pallas import tpu_sc as plsc

VOCAB = 100000
MIN_N = 2
MAX_N = 3
NUM_HEADS = 4
TBL = 100000
EDIM = 64
HID = 2048
KSIZE = 3
ORDERS = MAX_N - MIN_N + 1
MEMD = ORDERS * NUM_HEADS * EDIM
BATCH = 2
SEQ = 4096
BS = BATCH * SEQ

_rng = np.random.RandomState(1234)
_HASH_MULT = ((_rng.randint(1, 2**31 - 1, size=(NUM_HEADS, MAX_N)) * 2 + 1)
              % (2**32)).astype(np.uint32)

NW = 32
CHUNK = BS // NW
GSUB = 128
NSUB = CHUNK // GSUB


_RB = 4096


def _tp_body(tin_ref, out_ref):
    eye = jnp.eye(4 * EDIM, dtype=jnp.bfloat16)
    t = tin_ref[...].reshape(4 * EDIM, _RB).astype(jnp.bfloat16)
    tt = lax.dot_general(t, eye, (((0,), (0,)), ((), ())),
                         preferred_element_type=jnp.float32)
    b = lax.bitcast_convert_type(tt, jnp.uint32)
    packed = []
    for i in range(4):
        s = b[:, i * EDIM:(i + 1) * EDIM]
        w = (s[:, EDIM // 2:] & jnp.uint32(0xFFFF0000)) | (s[:, :EDIM // 2]
                                                          >> jnp.uint32(16))
        packed.append(lax.bitcast_convert_type(w, jnp.int32))
    out_ref[0] = jnp.concatenate(packed, axis=1)


def _tc_transpose_pad(tables_t):
    return pl.pallas_call(
        _tp_body,
        grid=(2, pl.cdiv(TBL, _RB)),
        in_specs=[pl.BlockSpec((4, EDIM, _RB), lambda g, j: (g, 0, j))],
        out_specs=pl.BlockSpec((1, _RB, 2 * EDIM), lambda g, j: (g, j, 0)),
        out_shape=jax.ShapeDtypeStruct((2, TBL, 2 * EDIM), jnp.int32),
    )(tables_t)


def _sc_gather_mem(tables_flat, idt, idm1, idm2):
    mesh = plsc.VectorSubcoreMesh(core_axis_name="c", subcore_axis_name="s")

    @functools.partial(
        pl.kernel,
        mesh=mesh,
        out_type=jax.ShapeDtypeStruct((ORDERS * NUM_HEADS, BS, 2 * EDIM),
                                      jnp.int32),
        scratch_types=[
            pltpu.VMEM((CHUNK,), jnp.int32),
            pltpu.VMEM((CHUNK,), jnp.int32),
            pltpu.VMEM((CHUNK,), jnp.int32),
            pltpu.VMEM((NSUB, GSUB), jnp.int32),
            pltpu.VMEM((CHUNK, 2 * EDIM), jnp.int32),
            pltpu.SemaphoreType.DMA,
        ],
    )
    def gk(tbl_hbm, idt_hbm, idm1_hbm, idm2_hbm, out_hbm,
           idt_v, idm1_v, idm2_v, idx_v, rows_v, sem):
        wid = lax.axis_index("s") * 2 + lax.axis_index("c")
        base = wid * CHUNK
        pltpu.sync_copy(idt_hbm.at[pl.ds(base, CHUNK)], idt_v)
        pltpu.sync_copy(idm1_hbm.at[pl.ds(base, CHUNK)], idm1_v)
        pltpu.sync_copy(idm2_hbm.at[pl.ds(base, CHUNK)], idm2_v)

        for o, n in enumerate(range(MIN_N, MAX_N + 1)):
            for h in range(NUM_HEADS):
                kk = o * NUM_HEADS + h
                m = _HASH_MULT[h]
                for v in range(CHUNK // 16):
                    sl = pl.ds(v * 16, 16)
                    t0 = plsc.bitcast(idt_v[sl], jnp.uint32)
                    t1 = plsc.bitcast(idm1_v[sl], jnp.uint32)
                    if n == 2:
                        acc = t1 * jnp.uint32(m[0]) + t0 * jnp.uint32(m[1])
                    else:
                        t2 = plsc.bitcast(idm2_v[sl], jnp.uint32)
                        acc = (t2 * jnp.uint32(m[0]) + t1 * jnp.uint32(m[1])
                               + t0 * jnp.uint32(m[2]))
                    acc = acc ^ (acc >> jnp.uint32(16))
                    acc = acc % jnp.uint32(TBL)
                    row = plsc.bitcast(acc, jnp.int32)
                    idx_v[v // (GSUB // 16), pl.ds((v % (GSUB // 16)) * 16, 16)] = row
                cps = [
                    pltpu.async_copy(
                        tbl_hbm.at[kk // 4].at[idx_v.at[c]],
                        rows_v.at[pl.ds(c * GSUB, GSUB)],
                        sem,
                    )
                    for c in range(NSUB)
                ]
                for cp in cps:
                    cp.wait()
                pltpu.sync_copy(rows_v, out_hbm.at[kk, pl.ds(base, CHUNK)])

    return gk(tables_flat, idt, idm1, idm2)


_TBLK = 512


def _tc_body(hs_ref, mem_ref, wg_ref, wv_ref, bg_ref, bv_ref, cw_ref, cb_ref,
             out_ref, carry_ref):
    j = pl.program_id(1)
    hs = hs_ref[0]
    cols = []
    for kk in range(ORDERS * NUM_HEADS):
        l = kk % 4
        x = mem_ref[kk][:, l * (EDIM // 2):(l + 1) * (EDIM // 2)]
        u = lax.bitcast_convert_type(x, jnp.uint32)
        lo = lax.bitcast_convert_type(u << jnp.uint32(16), jnp.float32)
        hi = lax.bitcast_convert_type(u & jnp.uint32(0xFFFF0000), jnp.float32)
        cols.append(lo)
        cols.append(hi)
    mem = jnp.concatenate(cols, axis=1)
    memh = mem.astype(jnp.bfloat16)
    keyv = jnp.dot(hs.astype(jnp.bfloat16), wg_ref[...],
                   preferred_element_type=jnp.float32) + bg_ref[...]
    dot = jnp.sum(keyv * mem, axis=1, keepdims=True) * (1.0 / np.sqrt(MEMD))
    alpha = 1.0 / (1.0 + jnp.exp(-dot))
    value = jnp.dot(memh, wv_ref[...],
                    preferred_element_type=jnp.float32) + bv_ref[...]
    gated = alpha * value
    prev = jnp.where(j == 0, 0.0, carry_ref[0:2])
    g_m1 = jnp.concatenate([prev[1:2], gated[:-1]], axis=0)
    g_m2 = jnp.concatenate([prev[0:2], gated[:-2]], axis=0)
    fused = (g_m2 * cw_ref[0:1] + g_m1 * cw_ref[1:2] + gated * cw_ref[2:3]
             + cb_ref[...])
    out_ref[0] = hs + fused
    carry_ref[0:2] = gated[_TBLK - 2:]


def _tc_dense(hs, mem3, Wg, bg, Wv, bv, conv_w, conv_b):
    grid = (BATCH, SEQ // _TBLK)
    return pl.pallas_call(
        _tc_body,
        grid=grid,
        in_specs=[
            pl.BlockSpec((1, _TBLK, HID), lambda b, j: (b, j, 0)),
            pl.BlockSpec((ORDERS * NUM_HEADS, _TBLK, 2 * EDIM),
                         lambda b, j: (0, b * (SEQ // _TBLK) + j, 0)),
            pl.BlockSpec((HID, MEMD), lambda b, j: (0, 0)),
            pl.BlockSpec((MEMD, HID), lambda b, j: (0, 0)),
            pl.BlockSpec((1, MEMD), lambda b, j: (0, 0)),
            pl.BlockSpec((1, HID), lambda b, j: (0, 0)),
            pl.BlockSpec((KSIZE, HID), lambda b, j: (0, 0)),
            pl.BlockSpec((1, HID), lambda b, j: (0, 0)),
        ],
        out_specs=pl.BlockSpec((1, _TBLK, HID), lambda b, j: (b, j, 0)),
        out_shape=jax.ShapeDtypeStruct((BATCH, SEQ, HID), jnp.float32),
        scratch_shapes=[pltpu.VMEM((8, HID), jnp.float32)],
        compiler_params=pltpu.CompilerParams(
            dimension_semantics=("arbitrary", "arbitrary"),
        ),
    )(hs, mem3, Wg.astype(jnp.bfloat16), Wv.astype(jnp.bfloat16),
      bg.reshape(1, MEMD), bv.reshape(1, HID),
      conv_w.T, conv_b.reshape(1, HID))


def kernel(hidden_states, input_ids, tables, Wg, bg, Wv, bv, conv_w, conv_b):
    ids = input_ids.astype(jnp.int32)
    idm1 = jnp.pad(ids, ((0, 0), (1, 0)))[:, :SEQ]
    idm2 = jnp.pad(ids, ((0, 0), (2, 0)))[:, :SEQ]
    tables_t = jnp.transpose(tables, (0, 2, 1))
    tables_pad = _tc_transpose_pad(tables_t)
    mem = _sc_gather_mem(tables_pad, ids.reshape(BS), idm1.reshape(BS),
                         idm2.reshape(BS))
    return _tc_dense(hidden_states, mem, Wg, bg, Wv, bv, conv_w, conv_b)
